# baseline (speedup 1.0000x reference)
import functools

import jax
import jax.numpy as jnp
from jax import lax
from jax.experimental import pallas as pl
from jax.experimental.pallas import tpu as pltpu
from jax.experimental.pallas import tpu_sc as plsc

NC = 2
NS = 16
LANES = 16
CHUNK = 256
SUB = CHUNK // 128


def _make_deg(n_chunks, n_out, n_pad):
  mesh = plsc.VectorSubcoreMesh(core_axis_name="c", subcore_axis_name="s")

  del n_out
  @functools.partial(
      pl.kernel,
      out_type=jax.ShapeDtypeStruct((NC, n_pad, 128), jnp.float32),
      mesh=mesh,
      scratch_types=[
          pltpu.VMEM((SUB, 128), jnp.int32),
          pltpu.VMEM((128, 128), jnp.float32),
          pltpu.VMEM_SHARED((n_pad, 128), jnp.float32),
      ],
  )
  def deg_kernel(dst_hbm, out_hbm, dst_v, ones_v, acc_sh):
    c = lax.axis_index("c")
    s = lax.axis_index("s")
    zero16 = jnp.zeros((LANES,), jnp.float32)
    one16 = jnp.ones((LANES,), jnp.float32)

    def fill(val):
      def frow(i, carry):
        for j in range(128 // LANES):
          ones_v[i, pl.ds(j * LANES, LANES)] = val
        return carry
      lax.fori_loop(0, 128, frow, 0)

    fill(zero16)

    rows_per_tile = n_pad // NS
    base = s * rows_per_tile
    done = 0
    while done < rows_per_tile:
      step = min(128, rows_per_tile - done)
      pltpu.sync_copy(ones_v.at[pl.ds(0, step)],
                      acc_sh.at[pl.ds(base + done, step)])
      done += step

    fill(one16)
    plsc.subcore_barrier()

    def body(i, carry):
      pltpu.sync_copy(dst_hbm.at[c].at[s].at[i], dst_v)
      for j in range(SUB):
        pltpu.sync_copy(ones_v, acc_sh.at[dst_v.at[j]], add=True)
      return carry

    lax.fori_loop(0, n_chunks, body, 0)
    plsc.subcore_barrier()

    out_rows = n_pad // NS
    pltpu.sync_copy(acc_sh.at[pl.ds(s * out_rows, out_rows)],
                    out_hbm.at[c].at[pl.ds(s * out_rows, out_rows)])

  return deg_kernel


def _make_segsum(n_chunks, n_out, n_pad):
  mesh = plsc.VectorSubcoreMesh(core_axis_name="c", subcore_axis_name="s")

  del n_out
  @functools.partial(
      pl.kernel,
      out_type=jax.ShapeDtypeStruct((NC, n_pad, 128), jnp.float32),
      mesh=mesh,
      scratch_types=[
          pltpu.VMEM((SUB, 128), jnp.int32),
          pltpu.VMEM((SUB, 128), jnp.int32),
          pltpu.VMEM((CHUNK, 128), jnp.float32),
          pltpu.VMEM_SHARED((n_pad, 128), jnp.float32),
          pltpu.SemaphoreType.DMA,
      ],
  )
  def segsum_kernel(table_hbm, src_hbm, dst_hbm, out_hbm,
                    src_v, dst_v, rows_v, acc_sh, sem):
    c = lax.axis_index("c")
    s = lax.axis_index("s")
    zero16 = jnp.zeros((LANES,), jnp.float32)

    def zrow(i, carry):
      for j in range(128 // LANES):
        rows_v[i, pl.ds(j * LANES, LANES)] = zero16
      return carry

    lax.fori_loop(0, CHUNK, zrow, 0)

    rows_per_tile = n_pad // NS
    base = s * rows_per_tile
    done = 0
    while done < rows_per_tile:
      step = min(CHUNK, rows_per_tile - done)
      pltpu.sync_copy(rows_v.at[pl.ds(0, step)],
                      acc_sh.at[pl.ds(base + done, step)])
      done += step
    plsc.subcore_barrier()

    def body(i, carry):
      pltpu.sync_copy(src_hbm.at[c].at[s].at[i], src_v)
      pltpu.sync_copy(dst_hbm.at[c].at[s].at[i], dst_v)
      cps = [
          pltpu.async_copy(table_hbm.at[src_v.at[j]],
                           rows_v.at[pl.ds(j * 128, 128)], sem)
          for j in range(SUB)
      ]
      for cp in cps:
        cp.wait()
      for j in range(SUB):
        pltpu.sync_copy(rows_v.at[pl.ds(j * 128, 128)],
                        acc_sh.at[dst_v.at[j]], add=True)
      return carry

    lax.fori_loop(0, n_chunks, body, 0)
    plsc.subcore_barrier()

    out_rows = n_pad // NS
    pltpu.sync_copy(acc_sh.at[pl.ds(s * out_rows, out_rows)],
                    out_hbm.at[c].at[pl.ds(s * out_rows, out_rows)])

  return segsum_kernel


def _stage_a(x, W1, deg_parts, block):
  n, d_in = x.shape
  d_hid = W1.shape[1]
  grid = n // block

  def body(x_ref, w_ref, dp_ref, h1s_ref, dinv_ref):
    d = dp_ref[0] + dp_ref[1] + 1.0
    dinv = lax.rsqrt(d)[:, 0:1]
    h = jnp.dot(x_ref[...], w_ref[...],
                preferred_element_type=jnp.float32,
                precision=lax.Precision.HIGHEST)
    h1s_ref[...] = h * dinv
    dinv_ref[...] = dinv

  return pl.pallas_call(
      body,
      grid=(grid,),
      in_specs=[
          pl.BlockSpec((block, d_in), lambda i: (i, 0)),
          pl.BlockSpec((d_in, d_hid), lambda i: (0, 0)),
          pl.BlockSpec((NC, block, 128), lambda i: (0, i, 0)),
      ],
      out_specs=[
          pl.BlockSpec((block, d_hid), lambda i: (i, 0)),
          pl.BlockSpec((block, 1), lambda i: (i, 0)),
      ],
      out_shape=[
          jax.ShapeDtypeStruct((n, d_hid), jnp.float32),
          jax.ShapeDtypeStruct((n, 1), jnp.float32),
      ],
  )(x, W1, deg_parts)


def _stage_b(acc1, h1s, dinv, b1, W2, block):
  n, d_hid = h1s.shape
  d_out = W2.shape[1]
  grid = n // block

  def body(acc_ref, h1s_ref, dinv_ref, b1_ref, w2_ref, h2s_ref):
    dv = dinv_ref[...]
    w2 = w2_ref[...]
    h2 = None
    for ci in range(NC):
      t = acc_ref[ci] + h1s_ref[:, ci * 128:(ci + 1) * 128]
      t = jnp.maximum(dv * t + b1_ref[:, ci * 128:(ci + 1) * 128], 0.0)
      p = jnp.dot(t, w2[ci * 128:(ci + 1) * 128, :],
                  preferred_element_type=jnp.float32,
                  precision=lax.Precision.HIGHEST)
      h2 = p if h2 is None else h2 + p
    h2s_ref[...] = h2 * dv

  return pl.pallas_call(
      body,
      grid=(grid,),
      in_specs=[
          pl.BlockSpec((NC, block, 128), lambda i: (0, i, 0)),
          pl.BlockSpec((block, d_hid), lambda i: (i, 0)),
          pl.BlockSpec((block, 1), lambda i: (i, 0)),
          pl.BlockSpec((1, d_hid), lambda i: (0, 0)),
          pl.BlockSpec((d_hid, d_out), lambda i: (0, 0)),
      ],
      out_specs=pl.BlockSpec((block, d_out), lambda i: (i, 0)),
      out_shape=jax.ShapeDtypeStruct((n, d_out), jnp.float32),
  )(acc1, h1s, dinv, b1, W2)


def _stage_c(acc2, h2s, dinv, b2, block):
  n, d_out = h2s.shape
  grid = n // block

  def body(acc_ref, h2s_ref, dinv_ref, b2_ref, out_ref):
    acc = acc_ref[0] + acc_ref[1]
    out_ref[...] = dinv_ref[...] * (acc + h2s_ref[...]) + b2_ref[...]

  return pl.pallas_call(
      body,
      grid=(grid,),
      in_specs=[
          pl.BlockSpec((NC, block, 128), lambda i: (0, i, 0)),
          pl.BlockSpec((block, d_out), lambda i: (i, 0)),
          pl.BlockSpec((block, 1), lambda i: (i, 0)),
          pl.BlockSpec((1, d_out), lambda i: (0, 0)),
      ],
      out_specs=pl.BlockSpec((block, d_out), lambda i: (i, 0)),
      out_shape=jax.ShapeDtypeStruct((n, d_out), jnp.float32),
  )(acc2, h2s, dinv, b2)


def kernel(x, edge_index, W1, b1, W2, b2):
  n = x.shape[0]
  e = edge_index.shape[1]
  per = NC * NS * CHUNK
  e_pad = -(-e // per) * per
  n_pad = -(-(n + 1) // (NS * 8)) * (NS * 8)
  pad = e_pad - e
  block = 1000

  src = edge_index[0]
  dst = edge_index[1]
  spare = n_pad - n
  src_p = jnp.concatenate([src, jnp.zeros((pad,), jnp.int32)])
  dst_p = jnp.concatenate(
      [dst, n + (jnp.arange(pad, dtype=jnp.int32) % spare)])

  deg_chunks = e_pad // per
  dst_deg = dst_p.reshape(NC, NS, deg_chunks, SUB, 128)
  deg_parts = _make_deg(deg_chunks, n, n_pad)(dst_deg)

  h1s, dinv = _stage_a(x, W1, deg_parts, block)

  l1_chunks = e_pad // (NS * CHUNK)
  src1 = jnp.stack([2 * src_p, 2 * src_p + 1]).reshape(NC, NS, l1_chunks,
                                                       SUB, 128)
  dst1 = jnp.stack([dst_p, dst_p]).reshape(NC, NS, l1_chunks, SUB, 128)
  acc1 = _make_segsum(l1_chunks, n, n_pad)(h1s.reshape(2 * n, 128),
                                           src1, dst1)

  h2s = _stage_b(acc1, h1s, dinv, b1.reshape(1, -1), W2, block)

  l2_chunks = e_pad // per
  src2 = src_p.reshape(NC, NS, l2_chunks, SUB, 128)
  dst2 = dst_p.reshape(NC, NS, l2_chunks, SUB, 128)
  acc2 = _make_segsum(l2_chunks, n, n_pad)(h2s, src2, dst2)

  return _stage_c(acc2, h2s, dinv, b2.reshape(1, -1), block)


# speedup vs baseline: 1.9735x; 1.9735x over previous
import functools

import jax
import jax.numpy as jnp
from jax import lax
from jax.experimental import pallas as pl
from jax.experimental.pallas import tpu as pltpu
from jax.experimental.pallas import tpu_sc as plsc

NC = 2
NS = 16
LANES = 16
CHUNK = 256
SUB = CHUNK // 128


def _make_deg(n_chunks, n_out, n_pad):
  mesh = plsc.VectorSubcoreMesh(core_axis_name="c", subcore_axis_name="s")

  del n_out
  @functools.partial(
      pl.kernel,
      out_type=jax.ShapeDtypeStruct((NC, n_pad, 128), jnp.float32),
      mesh=mesh,
      scratch_types=[
          pltpu.VMEM((SUB, 128), jnp.int32),
          pltpu.VMEM((128, 128), jnp.float32),
          pltpu.VMEM_SHARED((n_pad, 128), jnp.float32),
      ],
  )
  def deg_kernel(dst_hbm, out_hbm, dst_v, ones_v, acc_sh):
    c = lax.axis_index("c")
    s = lax.axis_index("s")
    zero16 = jnp.zeros((LANES,), jnp.float32)
    one16 = jnp.ones((LANES,), jnp.float32)

    def fill(val):
      def frow(i, carry):
        for j in range(128 // LANES):
          ones_v[i, pl.ds(j * LANES, LANES)] = val
        return carry
      lax.fori_loop(0, 128, frow, 0)

    fill(zero16)

    rows_per_tile = n_pad // NS
    base = s * rows_per_tile
    done = 0
    while done < rows_per_tile:
      step = min(128, rows_per_tile - done)
      pltpu.sync_copy(ones_v.at[pl.ds(0, step)],
                      acc_sh.at[pl.ds(base + done, step)])
      done += step

    fill(one16)
    plsc.subcore_barrier()

    def body(i, carry):
      pltpu.sync_copy(dst_hbm.at[c].at[s].at[i], dst_v)
      for j in range(SUB):
        pltpu.sync_copy(ones_v, acc_sh.at[dst_v.at[j]], add=True)
      return carry

    lax.fori_loop(0, n_chunks, body, 0)
    plsc.subcore_barrier()

    out_rows = n_pad // NS
    pltpu.sync_copy(acc_sh.at[pl.ds(s * out_rows, out_rows)],
                    out_hbm.at[c].at[pl.ds(s * out_rows, out_rows)])

  return deg_kernel


def _make_segsum(n_chunks, n_out, n_pad):
  mesh = plsc.VectorSubcoreMesh(core_axis_name="c", subcore_axis_name="s")

  del n_out
  @functools.partial(
      pl.kernel,
      out_type=jax.ShapeDtypeStruct((NC, n_pad, 128), jnp.float32),
      mesh=mesh,
      scratch_types=[
          pltpu.VMEM((SUB, 128), jnp.int32),
          pltpu.VMEM((SUB, 128), jnp.int32),
          pltpu.VMEM((CHUNK, 128), jnp.float32),
          pltpu.VMEM_SHARED((n_pad, 128), jnp.float32),
          pltpu.SemaphoreType.DMA,
      ],
  )
  def segsum_kernel(table_hbm, src_hbm, dst_hbm, out_hbm,
                    src_v, dst_v, rows_v, acc_sh, sem):
    c = lax.axis_index("c")
    s = lax.axis_index("s")
    zero16 = jnp.zeros((LANES,), jnp.float32)

    def zrow(i, carry):
      for j in range(128 // LANES):
        rows_v[i, pl.ds(j * LANES, LANES)] = zero16
      return carry

    lax.fori_loop(0, CHUNK, zrow, 0)

    rows_per_tile = n_pad // NS
    base = s * rows_per_tile
    done = 0
    while done < rows_per_tile:
      step = min(CHUNK, rows_per_tile - done)
      pltpu.sync_copy(rows_v.at[pl.ds(0, step)],
                      acc_sh.at[pl.ds(base + done, step)])
      done += step
    plsc.subcore_barrier()

    def body(i, carry):
      pltpu.sync_copy(src_hbm.at[c].at[s].at[i], src_v)
      pltpu.sync_copy(dst_hbm.at[c].at[s].at[i], dst_v)
      cps = [
          pltpu.async_copy(table_hbm.at[src_v.at[j]],
                           rows_v.at[pl.ds(j * 128, 128)], sem)
          for j in range(SUB)
      ]
      for cp in cps:
        cp.wait()
      for j in range(SUB):
        pltpu.sync_copy(rows_v.at[pl.ds(j * 128, 128)],
                        acc_sh.at[dst_v.at[j]], add=True)
      return carry

    lax.fori_loop(0, n_chunks, body, 0)
    plsc.subcore_barrier()

    out_rows = n_pad // NS
    pltpu.sync_copy(acc_sh.at[pl.ds(s * out_rows, out_rows)],
                    out_hbm.at[c].at[pl.ds(s * out_rows, out_rows)])

  return segsum_kernel


def _stage_a(x, W1, deg_parts, block):
  n, d_in = x.shape
  d_hid = W1.shape[1]
  grid = n // block

  def body(x_ref, w_ref, dp_ref, h1s_ref, dinv_ref):
    d = dp_ref[0] + dp_ref[1] + 1.0
    dinv = lax.rsqrt(d)[:, 0:1]
    h = jnp.dot(x_ref[...], w_ref[...],
                preferred_element_type=jnp.float32,
                precision=lax.Precision.HIGHEST)
    h1s_ref[...] = h * dinv
    dinv_ref[...] = dinv

  return pl.pallas_call(
      body,
      grid=(grid,),
      in_specs=[
          pl.BlockSpec((block, d_in), lambda i: (i, 0)),
          pl.BlockSpec((d_in, d_hid), lambda i: (0, 0)),
          pl.BlockSpec((NC, block, 128), lambda i: (0, i, 0)),
      ],
      out_specs=[
          pl.BlockSpec((block, d_hid), lambda i: (i, 0)),
          pl.BlockSpec((block, 1), lambda i: (i, 0)),
      ],
      out_shape=[
          jax.ShapeDtypeStruct((n, d_hid), jnp.float32),
          jax.ShapeDtypeStruct((n, 1), jnp.float32),
      ],
  )(x, W1, deg_parts)


def _stage_b(acc1, h1s, dinv, b1, W2, block):
  n, d_hid = h1s.shape
  d_out = W2.shape[1]
  grid = n // block

  def body(acc_ref, h1s_ref, dinv_ref, b1_ref, w2_ref, h2s_ref):
    dv = dinv_ref[...]
    w2 = w2_ref[...]
    h2 = None
    for ci in range(NC):
      t = acc_ref[ci] + h1s_ref[:, ci * 128:(ci + 1) * 128]
      t = jnp.maximum(dv * t + b1_ref[:, ci * 128:(ci + 1) * 128], 0.0)
      p = jnp.dot(t, w2[ci * 128:(ci + 1) * 128, :],
                  preferred_element_type=jnp.float32,
                  precision=lax.Precision.HIGHEST)
      h2 = p if h2 is None else h2 + p
    h2s_ref[...] = h2 * dv

  return pl.pallas_call(
      body,
      grid=(grid,),
      in_specs=[
          pl.BlockSpec((NC, block, 128), lambda i: (0, i, 0)),
          pl.BlockSpec((block, d_hid), lambda i: (i, 0)),
          pl.BlockSpec((block, 1), lambda i: (i, 0)),
          pl.BlockSpec((1, d_hid), lambda i: (0, 0)),
          pl.BlockSpec((d_hid, d_out), lambda i: (0, 0)),
      ],
      out_specs=pl.BlockSpec((block, d_out), lambda i: (i, 0)),
      out_shape=jax.ShapeDtypeStruct((n, d_out), jnp.float32),
  )(acc1, h1s, dinv, b1, W2)


def _stage_c(acc2, h2s, dinv, b2, block):
  n, d_out = h2s.shape
  grid = n // block

  def body(acc_ref, h2s_ref, dinv_ref, b2_ref, out_ref):
    acc = acc_ref[0] + acc_ref[1]
    out_ref[...] = dinv_ref[...] * (acc + h2s_ref[...]) + b2_ref[...]

  return pl.pallas_call(
      body,
      grid=(grid,),
      in_specs=[
          pl.BlockSpec((NC, block, 128), lambda i: (0, i, 0)),
          pl.BlockSpec((block, d_out), lambda i: (i, 0)),
          pl.BlockSpec((block, 1), lambda i: (i, 0)),
          pl.BlockSpec((1, d_out), lambda i: (0, 0)),
      ],
      out_specs=pl.BlockSpec((block, d_out), lambda i: (i, 0)),
      out_shape=jax.ShapeDtypeStruct((n, d_out), jnp.float32),
  )(acc2, h2s, dinv, b2)


def kernel(x, edge_index, W1, b1, W2, b2):
  n = x.shape[0]
  e = edge_index.shape[1]
  per = NC * NS * CHUNK
  e_pad = -(-e // per) * per
  n_pad = -(-(n + 1) // (NS * 8)) * (NS * 8)
  pad = e_pad - e
  block = 1000

  src = edge_index[0]
  dst = edge_index[1]
  spare = n_pad - n
  pad1 = pad // NS
  pad2 = pad // (NC * NS)
  pad_src1 = (jnp.arange(pad, dtype=jnp.int32) * 7919) % n
  pad_dst1 = n + jnp.arange(pad, dtype=jnp.int32) % spare
  src_t1 = jnp.concatenate(
      [src.reshape(NS, -1), pad_src1.reshape(NS, pad1)], axis=1)
  dst_t1 = jnp.concatenate(
      [dst.reshape(NS, -1), pad_dst1.reshape(NS, pad1)], axis=1)
  src_t2 = jnp.concatenate(
      [src.reshape(NC, NS, -1), pad_src1.reshape(NC, NS, pad2)], axis=2)
  dst_t2 = jnp.concatenate(
      [dst.reshape(NC, NS, -1), pad_dst1.reshape(NC, NS, pad2)], axis=2)

  deg_chunks = e_pad // per
  dst_deg = dst_t2.reshape(NC, NS, deg_chunks, SUB, 128)
  deg_parts = _make_deg(deg_chunks, n, n_pad)(dst_deg)

  h1s, dinv = _stage_a(x, W1, deg_parts, block)

  l1_chunks = e_pad // (NS * CHUNK)
  src1 = jnp.stack([2 * src_t1, 2 * src_t1 + 1]).reshape(NC, NS, l1_chunks,
                                                         SUB, 128)
  dst1 = jnp.stack([dst_t1, dst_t1]).reshape(NC, NS, l1_chunks, SUB, 128)
  acc1 = _make_segsum(l1_chunks, n, n_pad)(h1s.reshape(2 * n, 128),
                                           src1, dst1)

  h2s = _stage_b(acc1, h1s, dinv, b1.reshape(1, -1), W2, block)

  l2_chunks = e_pad // per
  src2 = src_t2.reshape(NC, NS, l2_chunks, SUB, 128)
  dst2 = dst_t2.reshape(NC, NS, l2_chunks, SUB, 128)
  acc2 = _make_segsum(l2_chunks, n, n_pad)(h2s, src2, dst2)

  return _stage_c(acc2, h2s, dinv, b2.reshape(1, -1), block)
